# Initial kernel scaffold; baseline (speedup 1.0000x reference)
import jax
import jax.numpy as jnp
from jax.experimental import pallas as pl


def kernel(emissions, tags, transitions):
    raise NotImplementedError("write your pallas kernel here")


# speedup vs baseline: 15.7076x; 15.7076x over previous
import functools

import jax
import jax.numpy as jnp
from jax import lax
from jax.experimental import pallas as pl
from jax.experimental.pallas import tpu as pltpu
from jax.experimental.pallas import tpu_sc as plsc

NUM_TAGS = 128
B = 64
T = 1024
NS = 16
L = 16
ROWS_PER_TILE = B // NS
WORDS_PER_TILE = ROWS_PER_TILE * T
CHUNKS = (T - 1) // L
TAIL = (T - 1) - CHUNKS * L

_mesh = plsc.VectorSubcoreMesh(
    core_axis_name="c", subcore_axis_name="s", num_cores=1, num_subcores=NS)


@functools.partial(
    pl.kernel,
    mesh=_mesh,
    out_type=(
        jax.ShapeDtypeStruct((NS, L), jnp.float32),
        jax.ShapeDtypeStruct((L,), jnp.float32),
    ),
    scratch_types=[
        pltpu.VMEM((NUM_TAGS * NUM_TAGS,), jnp.float32),
        pltpu.VMEM((WORDS_PER_TILE + L,), jnp.int32),
        pltpu.VMEM((L,), jnp.float32),
        pltpu.VMEM((NS, L), jnp.float32),
    ],
    compiler_params=pltpu.CompilerParams(needs_layout_passes=False),
)
def _crf_sc(trans_hbm, tags_hbm, parts_hbm, out_hbm, table_v, tags_v, stage_v,
            gbuf_v):
    w = lax.axis_index("s")
    pltpu.sync_copy(trans_hbm, table_v)
    pltpu.sync_copy(tags_hbm.at[pl.ds(w * WORDS_PER_TILE, WORDS_PER_TILE)],
                    tags_v.at[pl.ds(0, WORDS_PER_TILE)])
    tags_v[pl.ds(WORDS_PER_TILE, L)] = jnp.zeros((L,), jnp.int32)

    lanes = lax.iota(jnp.int32, L)
    acc = jnp.zeros((L,), jnp.float32)
    for r in range(ROWS_PER_TILE):
        base = r * T

        def chunk(c, a, base=base):
            off = base + c * L
            cur = tags_v[pl.ds(off, L)]
            nxt = tags_v[pl.ds(off + 1, L)]
            return a + plsc.load_gather(table_v, [cur * NUM_TAGS + nxt])

        acc = lax.fori_loop(0, CHUNKS, chunk, acc)
        off = base + CHUNKS * L
        cur = tags_v[pl.ds(off, L)]
        nxt = tags_v[pl.ds(off + 1, L)]
        vals = plsc.load_gather(table_v, [cur * NUM_TAGS + nxt])
        acc = acc + jnp.where(lanes < TAIL, vals, 0.0)

    stage_v[...] = acc
    pltpu.sync_copy(stage_v, parts_hbm.at[w])
    plsc.subcore_barrier()

    @pl.when(w == 0)
    def _finalize():
        pltpu.sync_copy(parts_hbm, gbuf_v)
        tot = gbuf_v[0, :]
        for i in range(1, NS):
            tot = tot + gbuf_v[i, :]
        stage_v[...] = plsc.cumsum(tot * (-1.0 / T))
        pltpu.sync_copy(stage_v, out_hbm)


def kernel(emissions, tags, transitions):
    del emissions
    tags_flat = tags.astype(jnp.int32).reshape(B * T)
    trans_flat = transitions.reshape(NUM_TAGS * NUM_TAGS)
    _, out = _crf_sc(trans_flat, tags_flat)
    return out[L - 1:L]


# speedup vs baseline: 16.3206x; 1.0390x over previous
import functools

import jax
import jax.numpy as jnp
from jax import lax
from jax.experimental import pallas as pl
from jax.experimental.pallas import tpu as pltpu
from jax.experimental.pallas import tpu_sc as plsc

NUM_TAGS = 128
B = 64
T = 1024
NS = 16
L = 16
ROWS_PER_TILE = B // NS
WORDS_PER_TILE = ROWS_PER_TILE * T
CHUNKS = (T - 1) // L
TAIL = (T - 1) - CHUNKS * L

_mesh = plsc.VectorSubcoreMesh(
    core_axis_name="c", subcore_axis_name="s", num_cores=1, num_subcores=NS)


@functools.partial(
    pl.kernel,
    mesh=_mesh,
    out_type=(
        jax.ShapeDtypeStruct((NS, L), jnp.float32),
        jax.ShapeDtypeStruct((L,), jnp.float32),
    ),
    scratch_types=[
        pltpu.VMEM((NUM_TAGS * NUM_TAGS,), jnp.float32),
        pltpu.VMEM((WORDS_PER_TILE + L,), jnp.int32),
        pltpu.VMEM((L,), jnp.float32),
        pltpu.VMEM((NS, L), jnp.float32),
        pltpu.SemaphoreType.DMA,
        pltpu.SemaphoreType.DMA,
    ],
    compiler_params=pltpu.CompilerParams(needs_layout_passes=False),
)
def _crf_sc(trans_hbm, tags_hbm, parts_hbm, out_hbm, table_v, tags_v, stage_v,
            gbuf_v, sem_a, sem_b):
    w = lax.axis_index("s")
    cp_a = pltpu.async_copy(trans_hbm, table_v, sem_a)
    cp_b = pltpu.async_copy(
        tags_hbm.at[pl.ds(w * WORDS_PER_TILE, WORDS_PER_TILE)],
        tags_v.at[pl.ds(0, WORDS_PER_TILE)], sem_b)
    tags_v[pl.ds(WORDS_PER_TILE, L)] = jnp.zeros((L,), jnp.int32)
    cp_a.wait()
    cp_b.wait()

    lanes = lax.iota(jnp.int32, L)
    acc = jnp.zeros((L,), jnp.float32)
    for r in range(ROWS_PER_TILE):
        base = r * T

        def chunk(c, a, base=base):
            off = base + c * L
            cur = tags_v[pl.ds(off, L)]
            nxt = tags_v[pl.ds(off + 1, L)]
            return a + plsc.load_gather(table_v, [cur * NUM_TAGS + nxt])

        acc = lax.fori_loop(0, CHUNKS // 7, lambda c, a: chunk(c * 7 + 6,
            chunk(c * 7 + 5, chunk(c * 7 + 4, chunk(c * 7 + 3, chunk(c * 7 + 2,
            chunk(c * 7 + 1, chunk(c * 7, a))))))), acc)
        off = base + CHUNKS * L
        cur = tags_v[pl.ds(off, L)]
        nxt = tags_v[pl.ds(off + 1, L)]
        vals = plsc.load_gather(table_v, [cur * NUM_TAGS + nxt])
        acc = acc + jnp.where(lanes < TAIL, vals, 0.0)

    stage_v[...] = acc
    pltpu.sync_copy(stage_v, parts_hbm.at[w])
    plsc.subcore_barrier()

    @pl.when(w == 0)
    def _finalize():
        pltpu.sync_copy(parts_hbm, gbuf_v)
        tot = gbuf_v[0, :]
        for i in range(1, NS):
            tot = tot + gbuf_v[i, :]
        stage_v[...] = plsc.cumsum(tot * (-1.0 / T))
        pltpu.sync_copy(stage_v, out_hbm)


def kernel(emissions, tags, transitions):
    del emissions
    tags_flat = tags.astype(jnp.int32).reshape(B * T)
    trans_flat = transitions.reshape(NUM_TAGS * NUM_TAGS)
    _, out = _crf_sc(trans_flat, tags_flat)
    return out[L - 1:L]


# speedup vs baseline: 16.7560x; 1.0267x over previous
import functools

import jax
import jax.numpy as jnp
from jax import lax
from jax.experimental import pallas as pl
from jax.experimental.pallas import tpu as pltpu
from jax.experimental.pallas import tpu_sc as plsc

NUM_TAGS = 128
B = 64
T = 1024
NS = 16
L = 16
ROWS_PER_TILE = B // NS
WORDS_PER_TILE = ROWS_PER_TILE * T
CHUNKS = T // L
NCHUNK = ROWS_PER_TILE * CHUNKS
TAIL = (T - 1) - (CHUNKS - 1) * L
TBL = NUM_TAGS * NUM_TAGS
DUMMY = TBL

_mesh = plsc.VectorSubcoreMesh(
    core_axis_name="c", subcore_axis_name="s", num_cores=1, num_subcores=NS)


@functools.partial(
    pl.kernel,
    mesh=_mesh,
    out_type=(
        jax.ShapeDtypeStruct((NS, L), jnp.float32),
        jax.ShapeDtypeStruct((L,), jnp.float32),
    ),
    scratch_types=[
        pltpu.VMEM((TBL + L,), jnp.float32),
        pltpu.VMEM((WORDS_PER_TILE + L,), jnp.int32),
        pltpu.VMEM((WORDS_PER_TILE,), jnp.int32),
        pltpu.VMEM((L,), jnp.float32),
        pltpu.VMEM((NS, L), jnp.float32),
        pltpu.SemaphoreType.DMA,
        pltpu.SemaphoreType.DMA,
    ],
    compiler_params=pltpu.CompilerParams(needs_layout_passes=False),
)
def _crf_sc(trans_hbm, tags_hbm, parts_hbm, out_hbm, table_v, tags_v, idx_v,
            stage_v, gbuf_v, sem_a, sem_b):
    w = lax.axis_index("s")
    cp_table = pltpu.async_copy(trans_hbm, table_v.at[pl.ds(0, TBL)], sem_a)
    cp_tags = pltpu.async_copy(
        tags_hbm.at[pl.ds(w * WORDS_PER_TILE, WORDS_PER_TILE)],
        tags_v.at[pl.ds(0, WORDS_PER_TILE)], sem_b)
    table_v[pl.ds(TBL, L)] = jnp.zeros((L,), jnp.float32)
    tags_v[pl.ds(WORDS_PER_TILE, L)] = jnp.zeros((L,), jnp.int32)
    lanes = lax.iota(jnp.int32, L)
    cp_tags.wait()

    def idx_chunk(k, _):
        off = k * L
        cur = tags_v[pl.ds(off, L)]
        nxt = tags_v[pl.ds(off + 1, L)]
        idx_v[pl.ds(off, L)] = cur * NUM_TAGS + nxt
        return 0

    lax.fori_loop(0, NCHUNK, idx_chunk, 0)
    for r in range(ROWS_PER_TILE):
        off = r * T + (CHUNKS - 1) * L
        v = idx_v[pl.ds(off, L)]
        idx_v[pl.ds(off, L)] = jnp.where(lanes < TAIL, v, DUMMY)
    cp_table.wait()

    def gather_chunk(k, a):
        return a + plsc.load_gather(table_v, [idx_v[pl.ds(k * L, L)]])

    def gather8(k, a):
        for j in range(8):
            a = gather_chunk(k * 8 + j, a)
        return a

    acc = lax.fori_loop(0, NCHUNK // 8, gather8, jnp.zeros((L,), jnp.float32))

    stage_v[...] = acc
    pltpu.sync_copy(stage_v, parts_hbm.at[w])
    plsc.subcore_barrier()

    @pl.when(w == 0)
    def _finalize():
        pltpu.sync_copy(parts_hbm, gbuf_v)
        tot = gbuf_v[0, :]
        for i in range(1, NS):
            tot = tot + gbuf_v[i, :]
        stage_v[...] = plsc.cumsum(tot * (-1.0 / T))
        pltpu.sync_copy(stage_v, out_hbm)


def kernel(emissions, tags, transitions):
    del emissions
    tags_flat = tags.astype(jnp.int32).reshape(B * T)
    trans_flat = transitions.reshape(TBL)
    _, out = _crf_sc(trans_flat, tags_flat)
    return out[L - 1:L]
